# baseline (speedup 1.0000x reference)
import functools

import jax
import jax.numpy as jnp
from jax import lax
from jax.experimental import pallas as pl
from jax.experimental.pallas import tpu as pltpu
from jax.experimental.pallas import tpu_sc as plsc

_VOCAB = 49408
_D = 768
_MAXLEN = 77
_B = 1024
_L = 77
_N = _B * _L
_LP = 80
_NP = _B * _LP
_NW = 32
_PER_W = _NP // _NW
_C = 16
_NCH = _PER_W // _C
_NBUF = 4
_LANES = 16
_HALF = _D // 2


def _body(tok_ids, pos_ids, tok_tab, ptab_hbm, out, tidx, pidx, ptab, tb,
          sg, so):
  wid = lax.axis_index("s") * 2 + lax.axis_index("c")
  base = wid * _PER_W

  pltpu.sync_copy(tok_ids.at[wid], tidx)
  pltpu.sync_copy(pos_ids.at[wid], pidx)
  pltpu.sync_copy(ptab_hbm, ptab)

  def gstart(g, slot):
    pltpu.async_copy(tok_tab.at[tidx.at[g]], tb.at[slot], sg.at[slot])

  def gwait(g, slot):
    pltpu.make_async_copy(tok_tab.at[tidx.at[g]], tb.at[slot],
                          sg.at[slot]).wait()

  def sstart(g, slot):
    pltpu.async_copy(tb.at[slot], out.at[pl.ds(base + g * _C, _C)],
                     so.at[slot])

  def swait(slot):
    pltpu.make_async_copy(tb.at[slot], out.at[pl.ds(base, _C)],
                          so.at[slot]).wait()

  gstart(0, 0)
  gstart(1, 1)
  col0 = lax.iota(jnp.int32, _LANES)
  himask = jnp.full((_LANES,), -65536, jnp.int32)

  @pl.loop(0, _NCH)
  def _chunk(g):
    slot = lax.rem(g, _NBUF)
    gwait(g, slot)

    @pl.when(g + 2 < _NCH)
    def _prefetch():
      nslot = lax.rem(g + 2, _NBUF)

      @pl.when(g >= 2)
      def _drain_prev_store():
        swait(nslot)

      gstart(g + 2, nslot)

    @pl.loop(0, _C, unroll=8)
    def _row(r):
      row_splat = plsc.load_gather(pidx.at[g],
                                   [jnp.full((_LANES,), r, jnp.int32)])
      rbase = row_splat * _HALF + col0
      for kk in range(_HALF // _LANES):
        off = kk * _LANES
        v = plsc.load_gather(ptab, [rbase + off])
        lo = lax.bitcast_convert_type(lax.shift_left(v, jnp.full((_LANES,), 16, jnp.int32)), jnp.float32)
        hi = lax.bitcast_convert_type(lax.bitwise_and(v, himask),
                                      jnp.float32)
        tb[slot, r, pl.ds(off, _LANES)] = (
            tb[slot, r, pl.ds(off, _LANES)] + lo)
        tb[slot, r, pl.ds(_HALF + off, _LANES)] = (
            tb[slot, r, pl.ds(_HALF + off, _LANES)] + hi)

    sstart(g, slot)

  for d in range(_NBUF):
    swait(lax.rem(_NCH - _NBUF + d, _NBUF))


def _pack_pos_table(position_table):
  lo = position_table[:, :_HALF].astype(jnp.bfloat16)
  hi = position_table[:, _HALF:].astype(jnp.bfloat16)
  lo16 = lax.bitcast_convert_type(lo, jnp.uint16).astype(jnp.uint32)
  hi16 = lax.bitcast_convert_type(hi, jnp.uint16).astype(jnp.uint32)
  packed = lax.bitwise_or(lo16, lax.shift_left(hi16, jnp.uint32(16)))
  return lax.bitcast_convert_type(packed, jnp.int32).reshape(-1)


@jax.jit
def kernel(input_ids, position_ids, token_table, position_table):
  ids_p = jnp.concatenate(
      [input_ids, input_ids[:, :_LP - _L]], axis=1).astype(jnp.int32)
  pos_p = jnp.concatenate(
      [position_ids, position_ids[:, :_LP - _L]], axis=1).astype(jnp.int32)
  tok = ids_p.reshape(_NW, _NCH, _C)
  pos = pos_p.reshape(_NW, _NCH, _C)

  mesh = plsc.VectorSubcoreMesh(core_axis_name="c", subcore_axis_name="s")
  kern = functools.partial(
      pl.kernel,
      out_type=jax.ShapeDtypeStruct((_NP, _D), jnp.float32),
      mesh=mesh,
      compiler_params=pltpu.CompilerParams(needs_layout_passes=False),
      scratch_types=[
          pltpu.VMEM((_NCH, _C), jnp.int32),
          pltpu.VMEM((_NCH, _C), jnp.int32),
          pltpu.VMEM((_MAXLEN * _HALF,), jnp.int32),
          pltpu.VMEM((_NBUF, _C, _D), jnp.float32),
          pltpu.SemaphoreType.DMA((_NBUF,)),
          pltpu.SemaphoreType.DMA((_NBUF,)),
      ],
  )(_body)
  flat = kern(tok, pos, token_table, _pack_pos_table(position_table))
  return flat.reshape(_B, _LP, _D)[:, :_L, :]


# speedup vs baseline: 1.7083x; 1.7083x over previous
import functools

import jax
import jax.numpy as jnp
from jax import lax
from jax.experimental import pallas as pl
from jax.experimental.pallas import tpu as pltpu
from jax.experimental.pallas import tpu_sc as plsc

_VOCAB = 49408
_D = 768
_MAXLEN = 77
_B = 1024
_L = 77
_N = _B * _L
_LP = 80
_NP = _B * _LP
_NW = 32
_PER_W = _NP // _NW
_C = 16
_NCH = _PER_W // _C
_NBUF = 4
_LANES = 16
_HALF = _D // 2


def _body(tok_ids, pos_ids, tok_tab, ptab_hbm, out, tidx, pidx, ptab, tb,
          sg, so):
  wid = lax.axis_index("s") * 2 + lax.axis_index("c")
  base = wid * _PER_W

  pltpu.sync_copy(tok_ids.at[wid], tidx)
  pltpu.sync_copy(pos_ids.at[wid], pidx)
  pltpu.sync_copy(ptab_hbm, ptab)

  def gstart(g, slot):
    pltpu.async_copy(tok_tab.at[tidx.at[g]], tb.at[slot], sg.at[slot])

  def gwait(g, slot):
    pltpu.make_async_copy(tok_tab.at[tidx.at[g]], tb.at[slot],
                          sg.at[slot]).wait()

  def sstart(g, slot):
    pltpu.async_copy(tb.at[slot], out.at[pl.ds(base + g * _C, _C)],
                     so.at[slot])

  def swait(slot):
    pltpu.make_async_copy(tb.at[slot], out.at[pl.ds(base, _C)],
                          so.at[slot]).wait()

  gstart(0, 0)
  gstart(1, 1)
  col0 = lax.iota(jnp.int32, _LANES)
  himask = jnp.full((_LANES,), -65536, jnp.int32)

  @pl.loop(0, _NCH)
  def _chunk(g):
    slot = lax.rem(g, _NBUF)
    gwait(g, slot)

    @pl.when(g + 2 < _NCH)
    def _prefetch():
      nslot = lax.rem(g + 2, _NBUF)

      @pl.when(g >= 2)
      def _drain_prev_store():
        swait(nslot)

      gstart(g + 2, nslot)

    @pl.loop(0, _C, unroll=2)
    def _row(r):
      row_splat = plsc.load_gather(pidx.at[g],
                                   [jnp.full((_LANES,), r, jnp.int32)])
      rbase = row_splat * _HALF + col0
      for kk in range(_HALF // _LANES):
        off = kk * _LANES
        v = plsc.load_gather(ptab, [rbase + off])
        lo = lax.bitcast_convert_type(lax.shift_left(v, jnp.full((_LANES,), 16, jnp.int32)), jnp.float32)
        hi = lax.bitcast_convert_type(lax.bitwise_and(v, himask),
                                      jnp.float32)
        tb[slot, r, pl.ds(off, _LANES)] = (
            tb[slot, r, pl.ds(off, _LANES)] + lo)
        tb[slot, r, pl.ds(_HALF + off, _LANES)] = (
            tb[slot, r, pl.ds(_HALF + off, _LANES)] + hi)

    sstart(g, slot)

  for d in range(_NBUF):
    swait(lax.rem(_NCH - _NBUF + d, _NBUF))


def _pack_pos_table(position_table):
  lo = position_table[:, :_HALF].astype(jnp.bfloat16)
  hi = position_table[:, _HALF:].astype(jnp.bfloat16)
  lo16 = lax.bitcast_convert_type(lo, jnp.uint16).astype(jnp.uint32)
  hi16 = lax.bitcast_convert_type(hi, jnp.uint16).astype(jnp.uint32)
  packed = lax.bitwise_or(lo16, lax.shift_left(hi16, jnp.uint32(16)))
  return lax.bitcast_convert_type(packed, jnp.int32).reshape(-1)


@jax.jit
def kernel(input_ids, position_ids, token_table, position_table):
  ids_p = jnp.concatenate(
      [input_ids, input_ids[:, :_LP - _L]], axis=1).astype(jnp.int32)
  pos_p = jnp.concatenate(
      [position_ids, position_ids[:, :_LP - _L]], axis=1).astype(jnp.int32)
  tok = ids_p.reshape(_NW, _NCH, _C)
  pos = pos_p.reshape(_NW, _NCH, _C)

  mesh = plsc.VectorSubcoreMesh(core_axis_name="c", subcore_axis_name="s")
  kern = functools.partial(
      pl.kernel,
      out_type=jax.ShapeDtypeStruct((_NP, _D), jnp.float32),
      mesh=mesh,
      compiler_params=pltpu.CompilerParams(needs_layout_passes=False),
      scratch_types=[
          pltpu.VMEM((_NCH, _C), jnp.int32),
          pltpu.VMEM((_NCH, _C), jnp.int32),
          pltpu.VMEM((_MAXLEN * _HALF,), jnp.int32),
          pltpu.VMEM((_NBUF, _C, _D), jnp.float32),
          pltpu.SemaphoreType.DMA((_NBUF,)),
          pltpu.SemaphoreType.DMA((_NBUF,)),
      ],
  )(_body)
  flat = kern(tok, pos, token_table, _pack_pos_table(position_table))
  return flat.reshape(_B, _LP, _D)[:, :_L, :]


# speedup vs baseline: 1.7163x; 1.0047x over previous
import functools

import jax
import jax.numpy as jnp
from jax import lax
from jax.experimental import pallas as pl
from jax.experimental.pallas import tpu as pltpu
from jax.experimental.pallas import tpu_sc as plsc

_VOCAB = 49408
_D = 768
_MAXLEN = 77
_B = 1024
_L = 77
_N = _B * _L
_LP = 80
_NP = _B * _LP
_NW = 32
_PER_W = _NP // _NW
_C = 16
_NCH = _PER_W // _C
_NBUF = 4
_LANES = 16
_HALF = _D // 2


def _body(tok_ids, pos_ids, tok_tab, ptab_hbm, out, tidx, pidx, ptab, tb,
          sg, so):
  wid = lax.axis_index("s") * 2 + lax.axis_index("c")
  base = wid * _PER_W

  pltpu.sync_copy(tok_ids.at[wid], tidx)
  pltpu.sync_copy(pos_ids.at[wid], pidx)
  pltpu.sync_copy(ptab_hbm, ptab)

  def gstart(g, slot):
    pltpu.async_copy(tok_tab.at[tidx.at[g]], tb.at[slot], sg.at[slot])

  def gwait(g, slot):
    pltpu.make_async_copy(tok_tab.at[tidx.at[g]], tb.at[slot],
                          sg.at[slot]).wait()

  def sstart(g, slot):
    pltpu.async_copy(tb.at[slot], out.at[pl.ds(base + g * _C, _C)],
                     so.at[slot])

  def swait(slot):
    pltpu.make_async_copy(tb.at[slot], out.at[pl.ds(base, _C)],
                          so.at[slot]).wait()

  gstart(0, 0)
  gstart(1, 1)
  col0 = lax.iota(jnp.int32, _LANES)
  himask = jnp.full((_LANES,), -65536, jnp.int32)

  @pl.loop(0, _NCH)
  def _chunk(g):
    slot = lax.rem(g, _NBUF)
    gwait(g, slot)

    @pl.when(g + 2 < _NCH)
    def _prefetch():
      nslot = lax.rem(g + 2, _NBUF)

      @pl.when(g >= 2)
      def _drain_prev_store():
        swait(nslot)

      gstart(g + 2, nslot)

    @pl.loop(0, _C, unroll=4)
    def _row(r):
      row_splat = plsc.load_gather(pidx.at[g],
                                   [jnp.full((_LANES,), r, jnp.int32)])
      rbase = row_splat * _HALF + col0
      for kk in range(_HALF // _LANES):
        off = kk * _LANES
        v = plsc.load_gather(ptab, [rbase + off])
        lo = lax.bitcast_convert_type(lax.shift_left(v, jnp.full((_LANES,), 16, jnp.int32)), jnp.float32)
        hi = lax.bitcast_convert_type(lax.bitwise_and(v, himask),
                                      jnp.float32)
        tb[slot, r, pl.ds(off, _LANES)] = (
            tb[slot, r, pl.ds(off, _LANES)] + lo)
        tb[slot, r, pl.ds(_HALF + off, _LANES)] = (
            tb[slot, r, pl.ds(_HALF + off, _LANES)] + hi)

    sstart(g, slot)

  for d in range(_NBUF):
    swait(lax.rem(_NCH - _NBUF + d, _NBUF))


def _pack_pos_table(position_table):
  lo = position_table[:, :_HALF].astype(jnp.bfloat16)
  hi = position_table[:, _HALF:].astype(jnp.bfloat16)
  lo16 = lax.bitcast_convert_type(lo, jnp.uint16).astype(jnp.uint32)
  hi16 = lax.bitcast_convert_type(hi, jnp.uint16).astype(jnp.uint32)
  packed = lax.bitwise_or(lo16, lax.shift_left(hi16, jnp.uint32(16)))
  return lax.bitcast_convert_type(packed, jnp.int32).reshape(-1)


@jax.jit
def kernel(input_ids, position_ids, token_table, position_table):
  ids_p = jnp.concatenate(
      [input_ids, input_ids[:, :_LP - _L]], axis=1).astype(jnp.int32)
  pos_p = jnp.concatenate(
      [position_ids, position_ids[:, :_LP - _L]], axis=1).astype(jnp.int32)
  tok = ids_p.reshape(_NW, _NCH, _C)
  pos = pos_p.reshape(_NW, _NCH, _C)

  mesh = plsc.VectorSubcoreMesh(core_axis_name="c", subcore_axis_name="s")
  kern = functools.partial(
      pl.kernel,
      out_type=jax.ShapeDtypeStruct((_NP, _D), jnp.float32),
      mesh=mesh,
      compiler_params=pltpu.CompilerParams(needs_layout_passes=False),
      scratch_types=[
          pltpu.VMEM((_NCH, _C), jnp.int32),
          pltpu.VMEM((_NCH, _C), jnp.int32),
          pltpu.VMEM((_MAXLEN * _HALF,), jnp.int32),
          pltpu.VMEM((_NBUF, _C, _D), jnp.float32),
          pltpu.SemaphoreType.DMA((_NBUF,)),
          pltpu.SemaphoreType.DMA((_NBUF,)),
      ],
  )(_body)
  flat = kern(tok, pos, token_table, _pack_pos_table(position_table))
  return flat.reshape(_B, _LP, _D)[:, :_L, :]
